# baseline (speedup 1.0000x reference)
import jax
import jax.numpy as jnp
from jax import lax
from jax.experimental import pallas as pl
from jax.experimental.pallas import tpu as pltpu
from jax.experimental.pallas import tpu_sc as plsc

N_VOCAB = 100000
N_HEADS = 8
N_FEATURES = 64
BATCH = 4096
HIST = 50

BTILE = 128
FT = N_FEATURES // 8
NSLOT = 4
L = 16


def _make_kernel():
    info = plsc.get_sparse_core_info()
    nc, ns = info.num_cores, info.num_subcores
    nw = nc * ns
    n_btiles = BATCH // BTILE
    assert n_btiles == nw

    mesh = plsc.VectorSubcoreMesh(core_axis_name="c", subcore_axis_name="s")

    def body(seq_hbm, tab_hbm, out_hbm, seq_v, base_v, idx_v, rows_v,
             tbuf_v, gsem, ssem):
        wid = lax.axis_index("s") * nc + lax.axis_index("c")
        seq_per_tile = BTILE * HIST
        pltpu.sync_copy(seq_hbm.at[pl.ds(wid * seq_per_tile, seq_per_tile)],
                        seq_v)
        iota = lax.iota(jnp.int32, L)

        def drain_gather(slot):
            pltpu.make_async_copy(
                tab_hbm.at[pl.ds(0, BTILE)], rows_v.at[slot],
                gsem.at[slot]).wait()

        def drain_stores(slot):
            for ft in range(FT):
                pltpu.make_async_copy(
                    out_hbm.at[0, 0, ft, wid], tbuf_v.at[slot, ft],
                    ssem.at[slot]).wait()

        def build_and_fire(slot, h):
            off = jnp.int32(h * N_VOCAB)
            for j in range(BTILE // L):
                idx_v[slot, pl.ds(j * L, L)] = base_v[pl.ds(j * L, L)] + off
            pltpu.async_copy(tab_hbm.at[idx_v.at[slot]], rows_v.at[slot],
                             gsem.at[slot])

        def retire(pslot, h_prev, t_prev, drain_s):
            drain_gather(pslot)
            if drain_s:
                drain_stores(pslot)

            def ft_body(ft, carry):
                for f8 in range(8):
                    f_vec = jnp.full((L,), ft * 8 + f8, jnp.int32)
                    for j in range(BTILE // L):
                        v = plsc.load_gather(
                            rows_v.at[pslot], [iota + (j * L), f_vec])
                        tbuf_v[pslot, ft, f8, pl.ds(j * L, L)] = v
                return carry

            lax.fori_loop(0, FT, ft_body, 0)
            for ft in range(FT):
                pltpu.async_copy(tbuf_v.at[pslot, ft],
                                 out_hbm.at[h_prev, t_prev, ft, wid],
                                 ssem.at[pslot])

        def t_body(t, carry):
            for j in range(BTILE // L):
                base_v[pl.ds(j * L, L)] = plsc.load_gather(
                    seq_v, [iota * HIST + (j * L * HIST + t)])
            for h in range(N_HEADS):
                slot = h % NSLOT
                build_and_fire(slot, h)
                pslot = (h - 1) % NSLOT
                if h == 0:
                    @pl.when(t > 0)
                    def _():
                        retire(pslot, N_HEADS - 1, t - 1, True)
                elif h <= 4:
                    drain_gather(pslot)

                    @pl.when(t > 0)
                    def _():
                        drain_stores(pslot)
                    retire_inner(pslot, h - 1, t)
                else:
                    retire(pslot, h - 1, t, True)
            return carry

        def retire_inner(pslot, h_prev, t_prev):
            def ft_body(ft, carry):
                for f8 in range(8):
                    f_vec = jnp.full((L,), ft * 8 + f8, jnp.int32)
                    for j in range(BTILE // L):
                        v = plsc.load_gather(
                            rows_v.at[pslot], [iota + (j * L), f_vec])
                        tbuf_v[pslot, ft, f8, pl.ds(j * L, L)] = v
                return carry

            lax.fori_loop(0, FT, ft_body, 0)
            for ft in range(FT):
                pltpu.async_copy(tbuf_v.at[pslot, ft],
                                 out_hbm.at[h_prev, t_prev, ft, wid],
                                 ssem.at[pslot])

        lax.fori_loop(0, HIST, t_body, 0)
        retire((N_HEADS - 1) % NSLOT, N_HEADS - 1, HIST - 1, True)
        for slot in range(NSLOT):
            drain_stores(slot)

    return pl.kernel(
        body,
        out_type=jax.ShapeDtypeStruct(
            (N_HEADS, HIST, FT, BATCH // BTILE, 8, BTILE), jnp.float32),
        mesh=mesh,
        scratch_types=[
            pltpu.VMEM((BTILE * HIST,), jnp.int32),
            pltpu.VMEM((BTILE,), jnp.int32),
            pltpu.VMEM((NSLOT, BTILE), jnp.int32),
            pltpu.VMEM((NSLOT, BTILE, N_FEATURES), jnp.float32),
            pltpu.VMEM((NSLOT, FT, 8, BTILE), jnp.float32),
            pltpu.SemaphoreType.DMA((NSLOT,)),
            pltpu.SemaphoreType.DMA((NSLOT,)),
        ],
        compiler_params=pltpu.CompilerParams(
            use_tc_tiling_on_sc=False, needs_layout_passes=False),
    )


def kernel(seq, tables):
    seq_flat = seq.reshape(-1).astype(jnp.int32)
    tab_flat = tables.reshape(N_HEADS * N_VOCAB, N_FEATURES)
    out6 = _make_kernel()(seq_flat, tab_flat)
    return out6.transpose(3, 5, 0, 1, 2, 4).reshape(
        BATCH, N_HEADS, HIST, N_FEATURES)


# speedup vs baseline: 1.2069x; 1.2069x over previous
import jax
import jax.numpy as jnp
from jax import lax
from jax.experimental import pallas as pl
from jax.experimental.pallas import tpu as pltpu
from jax.experimental.pallas import tpu_sc as plsc

N_VOCAB = 100000
N_HEADS = 8
N_FEATURES = 64
BATCH = 4096
HIST = 50

BTILE = 128
FT = N_FEATURES // 8
TILE_ELEMS = 8 * BTILE
NSLOT = 4
L = 16


def _make_kernel():
    info = plsc.get_sparse_core_info()
    nc, ns = info.num_cores, info.num_subcores
    nw = nc * ns
    assert BATCH // BTILE == nw

    mesh = plsc.VectorSubcoreMesh(core_axis_name="c", subcore_axis_name="s")

    def body(seq_hbm, tab_hbm, out_hbm, seq_v, base_v, idx_v, rows_v,
             tbuf_v, gsem, ssem):
        wid = lax.axis_index("s") * nc + lax.axis_index("c")
        seq_per_tile = BTILE * HIST
        pltpu.sync_copy(seq_hbm.at[pl.ds(wid * seq_per_tile, seq_per_tile)],
                        seq_v)
        iota = lax.iota(jnp.int32, L)
        scat = [iota * BTILE + (2048 * k) for k in range(4)]

        def drain_gather(slot):
            pltpu.make_async_copy(
                tab_hbm.at[pl.ds(0, BTILE)], rows_v.at[slot],
                gsem.at[slot]).wait()

        def drain_stores(slot):
            for ft in range(FT):
                pltpu.make_async_copy(
                    out_hbm.at[0, 0, ft, wid],
                    tbuf_v.at[slot, pl.ds(ft * TILE_ELEMS, TILE_ELEMS)],
                    ssem.at[slot]).wait()

        def build_and_fire(slot, h):
            off = jnp.int32(h * N_VOCAB)
            for j in range(BTILE // L):
                idx_v[slot, pl.ds(j * L, L)] = base_v[pl.ds(j * L, L)] + off
            pltpu.async_copy(tab_hbm.at[idx_v.at[slot]], rows_v.at[slot],
                             gsem.at[slot])

        def transpose_and_store(pslot, h_prev, t_prev):
            def c_body(c, carry):
                for bi in range(8):
                    b = c * 8 + bi
                    for k in range(4):
                        v = rows_v[pslot, b, pl.ds(k * L, L)]
                        plsc.store_scatter(
                            tbuf_v.at[pslot], [scat[k] + b], v)
                return carry

            lax.fori_loop(0, BTILE // 8, c_body, 0)
            for ft in range(FT):
                pltpu.async_copy(
                    tbuf_v.at[pslot, pl.ds(ft * TILE_ELEMS, TILE_ELEMS)],
                    out_hbm.at[h_prev, t_prev, ft, wid],
                    ssem.at[pslot])

        def t_body(t, carry):
            for j in range(BTILE // L):
                base_v[pl.ds(j * L, L)] = plsc.load_gather(
                    seq_v, [iota * HIST + (j * L * HIST + t)])
            for h in range(N_HEADS):
                slot = h % NSLOT
                build_and_fire(slot, h)
                pslot = (h - 1) % NSLOT
                if h == 0:
                    @pl.when(t > 0)
                    def _():
                        drain_gather(pslot)
                        drain_stores(pslot)
                        transpose_and_store(pslot, N_HEADS - 1, t - 1)
                elif h <= 4:
                    drain_gather(pslot)

                    @pl.when(t > 0)
                    def _():
                        drain_stores(pslot)
                    transpose_and_store(pslot, h - 1, t)
                else:
                    drain_gather(pslot)
                    drain_stores(pslot)
                    transpose_and_store(pslot, h - 1, t)
            return carry

        lax.fori_loop(0, HIST, t_body, 0)
        last = (N_HEADS - 1) % NSLOT
        drain_gather(last)
        drain_stores(last)
        transpose_and_store(last, N_HEADS - 1, HIST - 1)
        for slot in range(NSLOT):
            drain_stores(slot)

    return pl.kernel(
        body,
        out_type=jax.ShapeDtypeStruct(
            (N_HEADS, HIST, FT, BATCH // BTILE, TILE_ELEMS), jnp.float32),
        mesh=mesh,
        scratch_types=[
            pltpu.VMEM((BTILE * HIST,), jnp.int32),
            pltpu.VMEM((BTILE,), jnp.int32),
            pltpu.VMEM((NSLOT, BTILE), jnp.int32),
            pltpu.VMEM((NSLOT, BTILE, N_FEATURES), jnp.float32),
            pltpu.VMEM((NSLOT, FT * TILE_ELEMS), jnp.float32),
            pltpu.SemaphoreType.DMA((NSLOT,)),
            pltpu.SemaphoreType.DMA((NSLOT,)),
        ],
        compiler_params=pltpu.CompilerParams(
            use_tc_tiling_on_sc=False, needs_layout_passes=False),
    )


def kernel(seq, tables):
    seq_flat = seq.reshape(-1).astype(jnp.int32)
    tab_flat = tables.reshape(N_HEADS * N_VOCAB, N_FEATURES)
    out = _make_kernel()(seq_flat, tab_flat)
    return (out.reshape(N_HEADS, HIST, FT, BATCH // BTILE, 8, BTILE)
            .transpose(3, 5, 0, 1, 2, 4)
            .reshape(BATCH, N_HEADS, HIST, N_FEATURES))


# speedup vs baseline: 1.5585x; 1.2913x over previous
import jax
import jax.numpy as jnp
from jax import lax
from jax.experimental import pallas as pl
from jax.experimental.pallas import tpu as pltpu
from jax.experimental.pallas import tpu_sc as plsc

N_VOCAB = 100000
N_HEADS = 8
N_FEATURES = 64
BATCH = 4096
HIST = 50

BTILE = 128
FT = N_FEATURES // 8
TILE_ELEMS = 8 * BTILE
NSLOT = 4
L = 16


def _make_kernel():
    info = plsc.get_sparse_core_info()
    nc, ns = info.num_cores, info.num_subcores
    nw = nc * ns
    assert BATCH // BTILE == nw

    mesh = plsc.VectorSubcoreMesh(core_axis_name="c", subcore_axis_name="s")

    def body(seq_hbm, tab_hbm, out_hbm, seq_v, base_v, idx_v, rows_v,
             tbuf_v, gsem, ssem):
        wid = lax.axis_index("s") * nc + lax.axis_index("c")
        seq_per_tile = BTILE * HIST
        pltpu.sync_copy(seq_hbm.at[pl.ds(wid * seq_per_tile, seq_per_tile)],
                        seq_v)
        iota = lax.iota(jnp.int32, L)
        scat = [iota * BTILE + (2048 * k) for k in range(4)]

        def drain_gather(slot):
            pltpu.make_async_copy(
                tab_hbm.at[pl.ds(0, BTILE)], rows_v.at[slot],
                gsem.at[slot]).wait()

        def drain_stores(slot):
            for ft in range(FT):
                pltpu.make_async_copy(
                    out_hbm.at[0, 0, ft, wid],
                    tbuf_v.at[slot, pl.ds(ft * TILE_ELEMS, TILE_ELEMS)],
                    ssem.at[slot]).wait()

        def build_and_fire(slot, h):
            off = jnp.int32(h * N_VOCAB)
            for j in range(BTILE // L):
                idx_v[slot, pl.ds(j * L, L)] = base_v[pl.ds(j * L, L)] + off
            pltpu.async_copy(tab_hbm.at[idx_v.at[slot]], rows_v.at[slot],
                             gsem.at[slot])

        def transpose_and_store(pslot, h_prev, t_prev):
            @plsc.parallel_loop(0, BTILE, step=8, unroll=2)
            def _(c):
                for bi in range(8):
                    b = c + bi
                    for k in range(4):
                        v = rows_v[pslot, b, pl.ds(k * L, L)]
                        plsc.store_scatter(
                            tbuf_v.at[pslot], [scat[k] + b], v)
            for ft in range(FT):
                pltpu.async_copy(
                    tbuf_v.at[pslot, pl.ds(ft * TILE_ELEMS, TILE_ELEMS)],
                    out_hbm.at[h_prev, t_prev, ft, wid],
                    ssem.at[pslot])

        def t_body(t, carry):
            for j in range(BTILE // L):
                base_v[pl.ds(j * L, L)] = plsc.load_gather(
                    seq_v, [iota * HIST + (j * L * HIST + t)])
            for h in range(N_HEADS):
                slot = h % NSLOT
                build_and_fire(slot, h)
                pslot = (h - 1) % NSLOT
                if h == 0:
                    @pl.when(t > 0)
                    def _():
                        drain_gather(pslot)
                        drain_stores(pslot)
                        transpose_and_store(pslot, N_HEADS - 1, t - 1)
                elif h <= 4:
                    drain_gather(pslot)

                    @pl.when(t > 0)
                    def _():
                        drain_stores(pslot)
                    transpose_and_store(pslot, h - 1, t)
                else:
                    drain_gather(pslot)
                    drain_stores(pslot)
                    transpose_and_store(pslot, h - 1, t)
            return carry

        lax.fori_loop(0, HIST, t_body, 0)
        last = (N_HEADS - 1) % NSLOT
        drain_gather(last)
        drain_stores(last)
        transpose_and_store(last, N_HEADS - 1, HIST - 1)
        for slot in range(NSLOT):
            drain_stores(slot)

    return pl.kernel(
        body,
        out_type=jax.ShapeDtypeStruct(
            (N_HEADS, HIST, FT, BATCH // BTILE, TILE_ELEMS), jnp.float32),
        mesh=mesh,
        scratch_types=[
            pltpu.VMEM((BTILE * HIST,), jnp.int32),
            pltpu.VMEM((BTILE,), jnp.int32),
            pltpu.VMEM((NSLOT, BTILE), jnp.int32),
            pltpu.VMEM((NSLOT, BTILE, N_FEATURES), jnp.float32),
            pltpu.VMEM((NSLOT, FT * TILE_ELEMS), jnp.float32),
            pltpu.SemaphoreType.DMA((NSLOT,)),
            pltpu.SemaphoreType.DMA((NSLOT,)),
        ],
        compiler_params=pltpu.CompilerParams(
            use_tc_tiling_on_sc=False, needs_layout_passes=False),
    )


def kernel(seq, tables):
    seq_flat = seq.reshape(-1).astype(jnp.int32)
    tab_flat = tables.reshape(N_HEADS * N_VOCAB, N_FEATURES)
    out = _make_kernel()(seq_flat, tab_flat)
    return (out.reshape(N_HEADS, HIST, FT, BATCH // BTILE, 8, BTILE)
            .transpose(3, 5, 0, 1, 2, 4)
            .reshape(BATCH, N_HEADS, HIST, N_FEATURES))
